# baseline (speedup 1.0000x reference)
import functools
import jax
import jax.numpy as jnp
from jax import lax
from jax.experimental import pallas as pl
from jax.experimental.pallas import tpu as pltpu
from jax.experimental.pallas import tpu_sc as plsc

_N = 20000
_NP = 20480
_K = 300
_OSZ = 384
_CH = 2048
_TH = 0.7
_L = 16
_BIG = 2 ** 30


def _tree_max_pick(rm, ri):
    m = rm[0]
    p = ri[0]
    for t in range(1, _L):
        v = rm[t]
        idx = ri[t]
        b = (v > m) | ((v == m) & (idx < p))
        m = jnp.where(b, v, m)
        p = jnp.where(b, idx, p)
    return m, p


def _tree_sum(acc):
    s = acc[0]
    for t in range(1, _L):
        s = s + acc[t]
    return s


def _lane_select(vec, off):
    x = vec[0]
    for t in range(1, _L):
        x = jnp.where(off == t, vec[t], x)
    return x


def _sc_body(s_hbm, an_hbm, rg_hbm,
             keep_o, x_o, y_o, w_o, h_o,
             s0_v, s_v, ab0, ab1, ab2, ab3, rb0, rb1, rb2, rb3,
             pa_v, pr_v, keep_b, xb, yb, wb, hb, si, sf):
    cid = lax.axis_index("c")
    sid = lax.axis_index("s")
    img = sid * 2 + cid

    @pl.when(sid < 2)
    def _():
        iota = lax.broadcasted_iota(jnp.int32, (_L,), 0)
        neg = jnp.float32(-jnp.inf)
        negv = jnp.full((_L,), neg, jnp.float32)
        bigv = jnp.full((_L,), jnp.int32(_BIG), jnp.int32)
        i4 = img * 4
        zf = jnp.float32(0.0)

        sbase = pl.multiple_of(img * _NP, 8)
        pltpu.sync_copy(s_hbm.at[pl.ds(sbase, _NP)], s0_v)
        pltpu.sync_copy(s_hbm.at[pl.ds(sbase, _NP)], s_v)

        def mchunk(k, carry):
            rm, ri = carry
            v = s0_v[pl.ds(k * _L, _L)]
            idx = k * _L + iota
            better = (v > rm) | ((v == rm) & (idx < ri))
            return (jnp.where(better, v, rm), jnp.where(better, idx, ri))

        rm, ri = lax.fori_loop(0, _NP // _L, mchunk, (negv, bigv), unroll=8)
        m0, pick0 = _tree_max_pick(rm, ri)

        si[0] = jnp.int32(0)
        si[1] = pick0
        si[2] = jnp.int32(0)
        sf[0] = m0

        def fetch_decode(pick):
            base = pl.multiple_of(pick & ~jnp.int32(7), 8)
            off = pick - base
            planes = []
            for p in range(4):
                pltpu.sync_copy(
                    an_hbm.at[pl.ds(
                        pl.multiple_of((i4 + p) * _NP + base, 8), _L)], pa_v)
                pltpu.sync_copy(
                    rg_hbm.at[pl.ds(
                        pl.multiple_of((i4 + p) * _NP + base, 8), _L)], pr_v)
                planes.append((pa_v[...], pr_v[...]))
            (vxa, vox), (vya, voy), (vwa, vow), (vha, voh) = planes
            vx = vwa * vox + vxa
            vy = vha * voy + vya
            vw = vwa * jnp.exp(vow)
            vh = vha * jnp.exp(voh)
            zv = jnp.zeros((_L,), jnp.float32)
            va = (vw - vx) * (vh - vy)
            viw = jnp.maximum(vw - vx, zv)
            vih = jnp.maximum(vh - vy, zv)
            vint = viw * vih
            vsiou = vint / (va + va - vint + jnp.float32(1e-9))
            px = _lane_select(vx, off)
            py = _lane_select(vy, off)
            pw = _lane_select(vw, off)
            ph = _lane_select(vh, off)
            psiou = _lane_select(vsiou, off)
            return px, py, pw, ph, psiou

        def step(i, carry):
            stopv = si[0]
            mv = sf[0]

            @pl.when((stopv == 0) & (mv > neg))
            def _():
                pick = si[1]

                def rchunk(k, acc):
                    v0 = s0_v[pl.ds(k * _L, _L)]
                    idx = k * _L + iota
                    c = (v0 > mv) | ((v0 == mv) & (idx < pick))
                    return acc + jnp.where(c, jnp.int32(1), jnp.int32(0))

                acc = lax.fori_loop(0, _NP // _L, rchunk,
                                    jnp.zeros((_L,), jnp.int32), unroll=8)
                rank = _tree_sum(acc)

                px, py, pw, ph, siou = fetch_decode(pick)
                pa = (pw - px) * (ph - py)
                stuck = jnp.logical_not(siou > _TH)

                row = (i // _L) * _L
                lane = i % _L
                lm = iota == lane
                sl = pl.ds(row, _L)
                keep_b[sl] = jnp.where(lm, jnp.full((_L,), rank, jnp.int32),
                                       keep_b[sl])
                xb[sl] = jnp.where(lm, jnp.full((_L,), px, jnp.float32),
                                   xb[sl])
                yb[sl] = jnp.where(lm, jnp.full((_L,), py, jnp.float32),
                                   yb[sl])
                wb[sl] = jnp.where(lm, jnp.full((_L,), pw, jnp.float32),
                                   wb[sl])
                hb[sl] = jnp.where(lm, jnp.full((_L,), ph, jnp.float32),
                                   hb[sl])

                si[3] = rank
                sf[1] = px
                sf[2] = py
                sf[3] = pw
                sf[4] = ph

                @pl.when(i == 0)
                def _():
                    si[4] = rank
                    sf[5] = px
                    sf[6] = py
                    sf[7] = pw
                    sf[8] = ph

                @pl.when(stuck)
                def _():
                    si[0] = jnp.int32(1)

                @pl.when(jnp.logical_not(stuck))
                def _():
                    def big(j, carry2):
                        base = j * _CH
                        pltpu.sync_copy(
                            an_hbm.at[pl.ds(pl.multiple_of(
                                (i4 + 0) * _NP + base, 8), _CH)], ab0)
                        pltpu.sync_copy(
                            an_hbm.at[pl.ds(pl.multiple_of(
                                (i4 + 1) * _NP + base, 8), _CH)], ab1)
                        pltpu.sync_copy(
                            an_hbm.at[pl.ds(pl.multiple_of(
                                (i4 + 2) * _NP + base, 8), _CH)], ab2)
                        pltpu.sync_copy(
                            an_hbm.at[pl.ds(pl.multiple_of(
                                (i4 + 3) * _NP + base, 8), _CH)], ab3)
                        pltpu.sync_copy(
                            rg_hbm.at[pl.ds(pl.multiple_of(
                                (i4 + 0) * _NP + base, 8), _CH)], rb0)
                        pltpu.sync_copy(
                            rg_hbm.at[pl.ds(pl.multiple_of(
                                (i4 + 1) * _NP + base, 8), _CH)], rb1)
                        pltpu.sync_copy(
                            rg_hbm.at[pl.ds(pl.multiple_of(
                                (i4 + 2) * _NP + base, 8), _CH)], rb2)
                        pltpu.sync_copy(
                            rg_hbm.at[pl.ds(pl.multiple_of(
                                (i4 + 3) * _NP + base, 8), _CH)], rb3)

                        def inner(t, c2):
                            rm2, ri2 = c2
                            csl = pl.ds(t * _L, _L)
                            xa = ab0[csl]
                            ya = ab1[csl]
                            wa = ab2[csl]
                            ha = ab3[csl]
                            ox = rb0[csl]
                            oy = rb1[csl]
                            ow = rb2[csl]
                            oh = rb3[csl]
                            bx = wa * ox + xa
                            by = ha * oy + ya
                            bwv = wa * jnp.exp(ow)
                            bhv = ha * jnp.exp(oh)
                            ar = (bwv - bx) * (bhv - by)
                            xx1 = jnp.maximum(px, bx)
                            yy1 = jnp.maximum(py, by)
                            xx2 = jnp.minimum(pw, bwv)
                            yy2 = jnp.minimum(ph, bhv)
                            iw = jnp.maximum(xx2 - xx1, zf)
                            ih = jnp.maximum(yy2 - yy1, zf)
                            inter = iw * ih
                            iou = inter / (pa + ar - inter
                                           + jnp.float32(1e-9))
                            gsl = pl.ds(base + t * _L, _L)
                            sv = s_v[gsl]
                            ns = jnp.where(iou > _TH, neg, sv)
                            s_v[gsl] = ns
                            gidx = base + t * _L + iota
                            better = ((ns > rm2)
                                      | ((ns == rm2) & (gidx < ri2)))
                            return (jnp.where(better, ns, rm2),
                                    jnp.where(better, gidx, ri2))

                        return lax.fori_loop(0, _CH // _L, inner, carry2,
                                             unroll=4)

                    rm2, ri2 = lax.fori_loop(0, _NP // _CH, big,
                                             (negv, bigv))
                    m2, pick2 = _tree_max_pick(rm2, ri2)
                    sf[0] = m2
                    si[1] = pick2

                    @pl.when(m2 <= neg)
                    def _():
                        si[0] = jnp.int32(2)

                si[2] = i + 1

            return carry

        lax.fori_loop(0, _K, step, jnp.int32(0))

        stopv = si[0]
        use_f0 = stopv == 2
        itf = si[2]
        vr = jnp.full((_L,), jnp.where(use_f0, si[4], si[3]), jnp.int32)
        vx = jnp.full((_L,), jnp.where(use_f0, sf[5], sf[1]), jnp.float32)
        vy = jnp.full((_L,), jnp.where(use_f0, sf[6], sf[2]), jnp.float32)
        vw = jnp.full((_L,), jnp.where(use_f0, sf[7], sf[3]), jnp.float32)
        vh = jnp.full((_L,), jnp.where(use_f0, sf[8], sf[4]), jnp.float32)

        def fchunk(k, carry):
            sl = pl.ds(k * _L, _L)
            ge = (k * _L + iota) >= itf
            keep_b[sl] = jnp.where(ge, vr, keep_b[sl])
            xb[sl] = jnp.where(ge, vx, xb[sl])
            yb[sl] = jnp.where(ge, vy, yb[sl])
            wb[sl] = jnp.where(ge, vw, wb[sl])
            hb[sl] = jnp.where(ge, vh, hb[sl])
            return carry

        lax.fori_loop(0, _OSZ // _L, fchunk, jnp.int32(0))

        obase = pl.multiple_of(img * _OSZ, 8)
        pltpu.sync_copy(keep_b, keep_o.at[pl.ds(obase, _OSZ)])
        pltpu.sync_copy(xb, x_o.at[pl.ds(obase, _OSZ)])
        pltpu.sync_copy(yb, y_o.at[pl.ds(obase, _OSZ)])
        pltpu.sync_copy(wb, w_o.at[pl.ds(obase, _OSZ)])
        pltpu.sync_copy(hb, h_o.at[pl.ds(obase, _OSZ)])


def kernel(fg_scores, reg_scores, anchors, img_size):
    del img_size
    B = fg_scores.shape[0]
    pad = _NP - _N
    s_p = jnp.pad(fg_scores, ((0, 0), (0, pad)),
                  constant_values=-jnp.inf).reshape(B * _NP)
    an_p = jnp.pad(jnp.transpose(anchors, (0, 2, 1)),
                   ((0, 0), (0, 0), (0, pad))).reshape(B * 4 * _NP)
    rg_p = jnp.pad(jnp.transpose(reg_scores, (0, 2, 1)),
                   ((0, 0), (0, 0), (0, pad))).reshape(B * 4 * _NP)

    mesh = plsc.VectorSubcoreMesh(core_axis_name="c", subcore_axis_name="s")
    f32 = jnp.float32
    sck = functools.partial(
        pl.kernel,
        mesh=mesh,
        out_type=[jax.ShapeDtypeStruct((B * _OSZ,), jnp.int32)]
        + [jax.ShapeDtypeStruct((B * _OSZ,), f32)] * 4,
        scratch_types=[
            pltpu.VMEM((_NP,), f32),
            pltpu.VMEM((_NP,), f32),
            pltpu.VMEM((_CH,), f32),
            pltpu.VMEM((_CH,), f32),
            pltpu.VMEM((_CH,), f32),
            pltpu.VMEM((_CH,), f32),
            pltpu.VMEM((_CH,), f32),
            pltpu.VMEM((_CH,), f32),
            pltpu.VMEM((_CH,), f32),
            pltpu.VMEM((_CH,), f32),
            pltpu.VMEM((_L,), f32),
            pltpu.VMEM((_L,), f32),
            pltpu.VMEM((_OSZ,), jnp.int32),
            pltpu.VMEM((_OSZ,), f32),
            pltpu.VMEM((_OSZ,), f32),
            pltpu.VMEM((_OSZ,), f32),
            pltpu.VMEM((_OSZ,), f32),
            pltpu.SMEM((8,), jnp.int32),
            pltpu.SMEM((16,), f32),
        ],
    )(_sc_body)
    kr, kx, ky, kw, kh = sck(s_p, an_p, rg_p)
    kr = kr.reshape(B, _OSZ)
    kx = kx.reshape(B, _OSZ)
    ky = ky.reshape(B, _OSZ)
    kw = kw.reshape(B, _OSZ)
    kh = kh.reshape(B, _OSZ)
    keep = kr[:, :_K]
    sel = jnp.stack([kx[:, :_K], ky[:, :_K], kw[:, :_K], kh[:, :_K]],
                    axis=-1)
    return sel, keep
